# baseline (speedup 1.0000x reference)
import functools

import jax
import jax.numpy as jnp
from jax import lax
from jax.experimental import pallas as pl
from jax.experimental.pallas import tpu as pltpu
from jax.experimental.pallas import tpu_sc as plsc

N = 50000
E = 800000
HID = 64
HEADS = 4
CH = 16
G = 128
NT = 784
NBLK = 64
N_PAD = NBLK * NT
C = 128

_MESH = dict(core_axis_name="c", subcore_axis_name="s",
             num_cores=2, num_subcores=16)


def _embed_body(x_ref, w_ref, b_ref, o_ref):
    o_ref[...] = x_ref[...] @ w_ref[...] + b_ref[...]


def _aedge_body(ea_ref, m_ref, c_ref, o_ref):
    o_ref[...] = ea_ref[...] @ m_ref[...] + c_ref[...]


def _pre_body(h_ref, w_ref, v_ref, nt_ref, ad_ref):
    h = h_ref[...]
    proj = h @ w_ref[...]
    av = h @ v_ref[...]
    z = jnp.zeros((h.shape[0], 12), jnp.float32)
    nt_ref[...] = jnp.concatenate([proj, av[:, 0:4], z], axis=1)
    ad_ref[...] = av[:, 4:8]


def _epi_body(ot_ref, nt_ref, ad_ref, el_ref, hres_ref, bias_ref, g_ref,
              be_ref, o_ref, s1, s2):
    p = pl.program_id(0)
    i = pl.program_id(1)
    ot = ot_ref[...]
    ntab = nt_ref[...]
    num = ot[:, 0:64]
    den = ot[:, 64:68]
    hp = ntab[:, 0:64]
    asrc = ntab[:, 64:68]
    zl = asrc + ad_ref[...] + el_ref[...]
    exl = jnp.exp(jnp.maximum(zl, 0.2 * zl))
    denw = den + exl + 1e-16
    ones16 = jnp.ones((1, 16), jnp.float32)
    exw = jnp.concatenate([exl[:, h:h + 1] * ones16 for h in range(4)], axis=1)
    dw = jnp.concatenate([denw[:, h:h + 1] * ones16 for h in range(4)], axis=1)
    pre = (num + exw * hp) / dw + bias_ref[...]

    @pl.when(p == 0)
    def _():
        @pl.when(i == 0)
        def _():
            s1[...] = jnp.zeros_like(s1)
            s2[...] = jnp.zeros_like(s2)
        s1[...] += jnp.sum(pre, axis=0, keepdims=True)
        s2[...] += jnp.sum(pre * pre, axis=0, keepdims=True)

    @pl.when(p == 1)
    def _():
        mu = s1[...] / N
        var = s2[...] / N - mu * mu
        xn = (pre - mu) / jnp.sqrt(var + 1e-5)
        o_ref[...] = jnp.maximum(xn * g_ref[...] + be_ref[...], 0.0) + hres_ref[...]


def _head_body(gs_ref, gm_ref, cnt_ref, gf_ref, gcw, gcb, f1w, f1b, f2w, f2b,
               p1w, p1b, p2w, p2b, p3w, p3b, o_ref):
    cnt = cnt_ref[...]
    gsum = gs_ref[...]
    gmean = gsum / jnp.maximum(cnt, 1.0)
    gmax = jnp.where(cnt > 0, gm_ref[...], 0.0)
    graph = jnp.concatenate([gmean, gmax, gsum], axis=1)
    graph = jnp.maximum(graph @ gcw[...] + gcb[...], 0.0)
    glob = jnp.maximum(gf_ref[...] @ f1w[...] + f1b[...], 0.0)
    glob = glob @ f2w[...] + f2b[...]
    comb = jnp.concatenate([graph, glob], axis=1)
    o = jnp.maximum(comb @ p1w[...] + p1b[...], 0.0)
    o = jnp.maximum(o @ p2w[...] + p2b[...], 0.0)
    o_ref[...] = o @ p3w[...] + p3b[...]


@functools.cache
def _make_prep():
    @functools.partial(
        pl.kernel,
        out_type=jax.ShapeDtypeStruct((N_PAD * 16,), jnp.float32),
        mesh=plsc.VectorSubcoreMesh(**_MESH),
        scratch_types=[
            pltpu.VMEM((80,), jnp.int32),
            pltpu.VMEM((C + 16,), jnp.int32),
            pltpu.VMEM((C * 16 + 16,), jnp.float32),
            pltpu.VMEM((NT * 16,), jnp.float32),
        ],
    )
    def prep(dsts, ae16, eb, out, eb_v, dst_v, ae_v, acc):
        wid = lax.axis_index("s") * 2 + lax.axis_index("c")
        pltpu.sync_copy(eb, eb_v)
        zero16 = jnp.zeros((16,), jnp.float32)
        for sub in range(2):
            b = wid * 2 + sub
            nbase = b * NT

            def zbody(i, _):
                acc[pl.ds(i * 16, 16)] = zero16
                return 0

            lax.fori_loop(0, NT, zbody, 0)
            ebv = eb_v[pl.ds(b, 16)]
            e0 = ebv[0]
            e1 = ebv[1]

            def cbody(j, _):
                cb = j * C
                pltpu.sync_copy(dsts.at[pl.ds(cb, C)], dst_v.at[pl.ds(0, C)])
                pltpu.sync_copy(ae16.at[pl.ds(cb * 16, C * 16)],
                                ae_v.at[pl.ds(0, C * 16)])
                ks = jnp.maximum(e0 - cb, 0)
                ke = jnp.minimum(e1 - cb, C)

                def ebody(k, _):
                    dk = dst_v[pl.ds(k, 16)][0]
                    dl = jnp.clip(dk - nbase, 0, NT - 1)
                    cur = acc[pl.ds(dl * 16, 16)]
                    acc[pl.ds(dl * 16, 16)] = cur + ae_v[pl.ds(k * 16, 16)]
                    return 0

                lax.fori_loop(ks, ke, ebody, 0)
                return 0

            lax.fori_loop(e0 // C, (e1 + C - 1) // C, cbody, 0)
            pltpu.sync_copy(acc, out.at[pl.ds(nbase * 16, NT * 16)])

    return prep


@functools.cache
def _make_main(l):
    @functools.partial(
        pl.kernel,
        out_type=jax.ShapeDtypeStruct((N_PAD * 80,), jnp.float32),
        mesh=plsc.VectorSubcoreMesh(**_MESH),
        compiler_params=pltpu.CompilerParams(use_tc_tiling_on_sc=False),
        scratch_types=[
            pltpu.VMEM((80,), jnp.int32),
            pltpu.VMEM((2 * C,), jnp.int32),
            pltpu.VMEM((2 * C + 16,), jnp.int32),
            pltpu.VMEM((2 * C * 16 + 16,), jnp.float32),
            pltpu.VMEM((2 * C, 80), jnp.float32),
            pltpu.VMEM((NT * 4 + 16,), jnp.float32),
            pltpu.VMEM((NT * 80,), jnp.float32),
            pltpu.SemaphoreType.DMA,
            pltpu.SemaphoreType.DMA,
            pltpu.SemaphoreType.DMA,
            pltpu.SemaphoreType.DMA,
        ],
    )
    def main(node_tab, ad_tab, srcs, dsts, ae16, eb, out,
             eb_v, idx2, dst2, ae2, rows2, ad_v, acc,
             sga, sgb, sla, slb):
        wid = lax.axis_index("s") * 2 + lax.axis_index("c")
        pltpu.sync_copy(eb, eb_v)
        zero16 = jnp.zeros((16,), jnp.float32)
        sg = (sga, sgb)
        sl = (sla, slb)

        def start(j, o):
            cb = j * C
            pltpu.sync_copy(srcs.at[pl.ds(cb, C)], idx2.at[pl.ds(o * C, C)])
            pltpu.async_copy(node_tab.at[idx2.at[pl.ds(o * C, C)]],
                             rows2.at[pl.ds(o * C, C)], sg[o])
            pltpu.async_copy(dsts.at[pl.ds(cb, C)],
                             dst2.at[pl.ds(o * C, C)], sl[o])
            pltpu.async_copy(ae16.at[pl.ds(cb * 16, C * 16)],
                             ae2.at[pl.ds(o * C * 16, C * 16)], sl[o])

        def wait_slot(o):
            pltpu.make_async_copy(node_tab.at[idx2.at[pl.ds(o * C, C)]],
                                  rows2.at[pl.ds(o * C, C)], sg[o]).wait()
            pltpu.make_async_copy(dsts.at[pl.ds(0, C)],
                                  dst2.at[pl.ds(o * C, C)], sl[o]).wait()
            pltpu.make_async_copy(ae16.at[pl.ds(0, C * 16)],
                                  ae2.at[pl.ds(o * C * 16, C * 16)], sl[o]).wait()

        for sub in range(2):
            b = wid * 2 + sub
            nbase = b * NT
            pltpu.sync_copy(ad_tab.at[pl.ds(nbase * 4, NT * 4)],
                            ad_v.at[pl.ds(0, NT * 4)])

            def zacc(i, _):
                acc[pl.ds(i * 16, 16)] = zero16
                return 0

            lax.fori_loop(0, NT * 5, zacc, 0)
            ebv = eb_v[pl.ds(b, 16)]
            e0 = ebv[0]
            e1 = ebv[1]
            c0 = e0 // C
            c1 = (e1 + C - 1) // C

            def process(j, o):
                cb = j * C
                ks = jnp.maximum(e0 - cb, 0)
                ke = jnp.minimum(e1 - cb, C)

                def ebody(k, _):
                    dk = dst2[pl.ds(o * C + k, 16)][0]
                    dl = jnp.clip(dk - nbase, 0, NT - 1)
                    asr = rows2[o * C + k, pl.ds(64, 16)]
                    adv = ad_v[pl.ds(dl * 4, 16)]
                    aev = ae2[pl.ds((o * C + k) * 16 + 4 * l, 16)]
                    a = asr + adv + aev
                    ex = jnp.exp(jnp.maximum(a, 0.2 * a))
                    dbase = dl * 80
                    cur = acc[pl.ds(dbase + 64, 16)]
                    acc[pl.ds(dbase + 64, 16)] = cur + ex
                    for h in range(HEADS):
                        exs = ex[h]
                        row = rows2[o * C + k, pl.ds(h * 16, 16)]
                        c2 = acc[pl.ds(dbase + h * 16, 16)]
                        acc[pl.ds(dbase + h * 16, 16)] = c2 + exs * row
                    return 0

                lax.fori_loop(ks, ke, ebody, 0)

            @pl.when(c1 > c0)
            def _():
                start(c0, 0)

            def pair_body(i, _):
                j0 = c0 + 2 * i
                j1 = j0 + 1

                @pl.when(j1 < c1)
                def _():
                    start(j1, 1)

                wait_slot(0)
                process(j0, 0)

                @pl.when(j1 + 1 < c1)
                def _():
                    start(j1 + 1, 0)

                @pl.when(j1 < c1)
                def _():
                    wait_slot(1)
                    process(j1, 1)

                return 0

            lax.fori_loop(0, (c1 - c0 + 1) // 2, pair_body, 0)
            pltpu.sync_copy(acc, out.at[pl.ds(nbase * 80, NT * 80)])

    return main


@functools.cache
def _make_pool():
    @functools.partial(
        pl.kernel,
        out_type=[jax.ShapeDtypeStruct((G * 64,), jnp.float32),
                  jax.ShapeDtypeStruct((G * 64,), jnp.float32)],
        mesh=plsc.VectorSubcoreMesh(**_MESH),
        scratch_types=[
            pltpu.VMEM((144,), jnp.int32),
            pltpu.VMEM((C * 64,), jnp.float32),
            pltpu.VMEM((64,), jnp.float32),
            pltpu.VMEM((64,), jnp.float32),
        ],
    )
    def pool(h_pad, gb, gsum, gmax, gb_v, h_v, ssum, smax):
        wid = lax.axis_index("s") * 2 + lax.axis_index("c")
        pltpu.sync_copy(gb, gb_v)
        zero16 = jnp.zeros((16,), jnp.float32)
        ninf16 = jnp.full((16,), -1e30, jnp.float32)
        for gg in range(4):
            g = wid * 4 + gg
            gbv = gb_v[pl.ds(g, 16)]
            s = gbv[0]
            e = gbv[1]
            for h in range(4):
                ssum[pl.ds(h * 16, 16)] = zero16
                smax[pl.ds(h * 16, 16)] = ninf16

            def cbody(j, _):
                cb = j * C
                pltpu.sync_copy(h_pad.at[pl.ds(cb * 64, C * 64)], h_v)
                ks = jnp.maximum(s - cb, 0)
                ke = jnp.minimum(e - cb, C)

                def nbody(k, _):
                    for h in range(4):
                        r = h_v[pl.ds(k * 64 + h * 16, 16)]
                        cs = ssum[pl.ds(h * 16, 16)]
                        ssum[pl.ds(h * 16, 16)] = cs + r
                        cm = smax[pl.ds(h * 16, 16)]
                        smax[pl.ds(h * 16, 16)] = jnp.maximum(cm, r)
                    return 0

                lax.fori_loop(ks, ke, nbody, 0)
                return 0

            lax.fori_loop(s // C, (e + C - 1) // C, cbody, 0)
            pltpu.sync_copy(ssum, gsum.at[pl.ds(g * 64, 64)])
            pltpu.sync_copy(smax, gmax.at[pl.ds(g * 64, 64)])

    return pool


def kernel(x, edge_index, edge_attr, batch, global_features, params):
    src = edge_index[0]
    dst = edge_index[1]
    perm = jnp.argsort(dst)
    dst_s = jnp.take(dst, perm)
    src_s = jnp.take(src, perm)
    ea_s = jnp.take(edge_attr, perm, axis=0)

    nb = jnp.searchsorted(
        dst_s, jnp.arange(N_PAD + 1, dtype=jnp.int32)).astype(jnp.int32)
    deg = (nb[1:N + 1] - nb[:N]).astype(jnp.float32)
    eb = jnp.pad(nb[jnp.arange(NBLK + 1) * NT], (0, 80 - (NBLK + 1)),
                 constant_values=E)

    Vs_list, Vd_list, M_list, c_list = [], [], [], []
    for lp in params['layers']:
        Wr = lp['W'].reshape(HID, HEADS, CH)
        Vs_list.append(jnp.einsum('dhc,hc->dh', Wr, lp['att_src']))
        Vd_list.append(jnp.einsum('dhc,hc->dh', Wr, lp['att_dst']))
        Wer = lp['W_edge'].reshape(HID, HEADS, CH)
        Ve = jnp.einsum('dhc,hc->dh', Wer, lp['att_edge'])
        M_list.append(params['edge_W'] @ Ve)
        c_list.append(params['edge_b'] @ Ve)
    Mcat8 = jnp.pad(jnp.concatenate(M_list, axis=1), ((0, 5), (0, 0)))
    ccat = jnp.concatenate(c_list).reshape(1, 16)

    ea8 = jnp.pad(ea_s, ((0, 0), (0, 5)))
    ae16 = pl.pallas_call(
        _aedge_body,
        grid=(250,),
        in_specs=[pl.BlockSpec((3200, 8), lambda i: (i, 0)),
                  pl.BlockSpec((8, 16), lambda i: (0, 0)),
                  pl.BlockSpec((1, 16), lambda i: (0, 0))],
        out_specs=pl.BlockSpec((3200, 16), lambda i: (i, 0)),
        out_shape=jax.ShapeDtypeStruct((E, 16), jnp.float32),
    )(ea8, Mcat8, ccat)

    esum = _make_prep()(dst_s, ae16.reshape(-1), eb).reshape(N_PAD, 16)
    el16 = esum[:N] / jnp.maximum(deg, 1.0)[:, None]

    x8 = jnp.pad(x, ((0, 0), (0, 1)))
    nw8 = jnp.pad(params['node_W'], ((0, 1), (0, 0)))
    h = pl.pallas_call(
        _embed_body,
        grid=(25,),
        in_specs=[pl.BlockSpec((2000, 8), lambda i: (i, 0)),
                  pl.BlockSpec((8, 64), lambda i: (0, 0)),
                  pl.BlockSpec((1, 64), lambda i: (0, 0))],
        out_specs=pl.BlockSpec((2000, 64), lambda i: (i, 0)),
        out_shape=jax.ShapeDtypeStruct((N, 64), jnp.float32),
    )(x8, nw8, params['node_b'].reshape(1, 64))

    for l, lp in enumerate(params['layers']):
        Vsd = jnp.concatenate([Vs_list[l], Vd_list[l]], axis=1)
        node_tab, a_dst = pl.pallas_call(
            _pre_body,
            grid=(25,),
            in_specs=[pl.BlockSpec((2000, 64), lambda i: (i, 0)),
                      pl.BlockSpec((64, 64), lambda i: (0, 0)),
                      pl.BlockSpec((64, 8), lambda i: (0, 0))],
            out_specs=[pl.BlockSpec((2000, 80), lambda i: (i, 0)),
                       pl.BlockSpec((2000, 4), lambda i: (i, 0))],
            out_shape=[jax.ShapeDtypeStruct((N, 80), jnp.float32),
                       jax.ShapeDtypeStruct((N, 4), jnp.float32)],
        )(h, lp['W'], Vsd)
        ad_pad = jnp.pad(a_dst, ((0, N_PAD - N), (0, 0)))
        out_tab = _make_main(l)(node_tab, ad_pad.reshape(-1), src_s, dst_s,
                                ae16.reshape(-1), eb).reshape(N_PAD, 80)
        el_l = el16[:, 4 * l:4 * l + 4]
        h = pl.pallas_call(
            _epi_body,
            grid=(2, 25),
            in_specs=[pl.BlockSpec((2000, 80), lambda p, i: (i, 0)),
                      pl.BlockSpec((2000, 80), lambda p, i: (i, 0)),
                      pl.BlockSpec((2000, 4), lambda p, i: (i, 0)),
                      pl.BlockSpec((2000, 4), lambda p, i: (i, 0)),
                      pl.BlockSpec((2000, 64), lambda p, i: (i, 0)),
                      pl.BlockSpec((1, 64), lambda p, i: (0, 0)),
                      pl.BlockSpec((1, 64), lambda p, i: (0, 0)),
                      pl.BlockSpec((1, 64), lambda p, i: (0, 0))],
            out_specs=pl.BlockSpec((2000, 64), lambda p, i: (i, 0)),
            out_shape=jax.ShapeDtypeStruct((N, 64), jnp.float32),
            scratch_shapes=[pltpu.VMEM((1, 64), jnp.float32),
                            pltpu.VMEM((1, 64), jnp.float32)],
        )(out_tab[:N], node_tab, a_dst, el_l, h,
          lp['bias'].reshape(1, 64), lp['bn_gamma'].reshape(1, 64),
          lp['bn_beta'].reshape(1, 64))

    gb = jnp.searchsorted(
        batch, jnp.arange(G + 1, dtype=jnp.int32)).astype(jnp.int32)
    gb_pad = jnp.pad(gb, (0, 144 - (G + 1)), constant_values=N)
    counts = (gb[1:] - gb[:-1]).astype(jnp.float32).reshape(G, 1)
    h_pad = jnp.pad(h, ((0, N_PAD - N), (0, 0)))
    gsum3, gmax3 = _make_pool()(h_pad.reshape(-1), gb_pad)

    out = pl.pallas_call(
        _head_body,
        out_shape=jax.ShapeDtypeStruct((G, 5), jnp.float32),
    )(gsum3.reshape(G, 64), gmax3.reshape(G, 64), counts, global_features,
      params['gc_W'], params['gc_b'].reshape(1, 64),
      params['gf1_W'], params['gf1_b'].reshape(1, 32),
      params['gf2_W'], params['gf2_b'].reshape(1, 32),
      params['p1_W'], params['p1_b'].reshape(1, 64),
      params['p2_W'], params['p2_b'].reshape(1, 32),
      params['p3_W'], params['p3_b'].reshape(1, 5))
    return out


# speedup vs baseline: 1.1364x; 1.1364x over previous
import functools

import jax
import jax.numpy as jnp
from jax import lax
from jax.experimental import pallas as pl
from jax.experimental.pallas import tpu as pltpu
from jax.experimental.pallas import tpu_sc as plsc

N = 50000
E = 800000
HID = 64
HEADS = 4
CH = 16
G = 128
NT = 784
NBLK = 64
N_PAD = NBLK * NT
C = 128

_MESH = dict(core_axis_name="c", subcore_axis_name="s",
             num_cores=2, num_subcores=16)


def _embed_body(x_ref, w_ref, b_ref, o_ref):
    o_ref[...] = x_ref[...] @ w_ref[...] + b_ref[...]


def _aedge_body(pk_ref, m_ref, c_ref, o_ref):
    ea = jax.lax.bitcast_convert_type(pk_ref[...], jnp.float32)
    o_ref[...] = ea @ m_ref[...] + c_ref[...]


def _pre_body(h_ref, w_ref, v_ref, nt_ref, ad_ref):
    h = h_ref[...]
    proj = h @ w_ref[...]
    av = h @ v_ref[...]
    z = jnp.zeros((h.shape[0], 12), jnp.float32)
    nt_ref[...] = jnp.concatenate([proj, av[:, 0:4], z], axis=1)
    ad_ref[...] = av[:, 4:8]


def _epi_body(ot_ref, nt_ref, ad_ref, el_ref, hres_ref, bias_ref, g_ref,
              be_ref, o_ref, s1, s2):
    p = pl.program_id(0)
    i = pl.program_id(1)
    ot = ot_ref[...]
    ntab = nt_ref[...]
    num = ot[:, 0:64]
    den = ot[:, 64:68]
    hp = ntab[:, 0:64]
    asrc = ntab[:, 64:68]
    zl = asrc + ad_ref[...] + el_ref[...]
    exl = jnp.exp(jnp.maximum(zl, 0.2 * zl))
    denw = den + exl + 1e-16
    ones16 = jnp.ones((1, 16), jnp.float32)
    exw = jnp.concatenate([exl[:, h:h + 1] * ones16 for h in range(4)], axis=1)
    dw = jnp.concatenate([denw[:, h:h + 1] * ones16 for h in range(4)], axis=1)
    pre = (num + exw * hp) / dw + bias_ref[...]

    @pl.when(p == 0)
    def _():
        @pl.when(i == 0)
        def _():
            s1[...] = jnp.zeros_like(s1)
            s2[...] = jnp.zeros_like(s2)
        s1[...] += jnp.sum(pre, axis=0, keepdims=True)
        s2[...] += jnp.sum(pre * pre, axis=0, keepdims=True)

    @pl.when(p == 1)
    def _():
        mu = s1[...] / N
        var = s2[...] / N - mu * mu
        xn = (pre - mu) / jnp.sqrt(var + 1e-5)
        o_ref[...] = jnp.maximum(xn * g_ref[...] + be_ref[...], 0.0) + hres_ref[...]


def _head_body(gs_ref, gm_ref, cnt_ref, gf_ref, gcw, gcb, f1w, f1b, f2w, f2b,
               p1w, p1b, p2w, p2b, p3w, p3b, o_ref):
    cnt = cnt_ref[...]
    gsum = gs_ref[...]
    gmean = gsum / jnp.maximum(cnt, 1.0)
    gmax = jnp.where(cnt > 0, gm_ref[...], 0.0)
    graph = jnp.concatenate([gmean, gmax, gsum], axis=1)
    graph = jnp.maximum(graph @ gcw[...] + gcb[...], 0.0)
    glob = jnp.maximum(gf_ref[...] @ f1w[...] + f1b[...], 0.0)
    glob = glob @ f2w[...] + f2b[...]
    comb = jnp.concatenate([graph, glob], axis=1)
    o = jnp.maximum(comb @ p1w[...] + p1b[...], 0.0)
    o = jnp.maximum(o @ p2w[...] + p2b[...], 0.0)
    o_ref[...] = o @ p3w[...] + p3b[...]


@functools.cache
def _make_prep():
    @functools.partial(
        pl.kernel,
        out_type=jax.ShapeDtypeStruct((N_PAD * 16,), jnp.float32),
        mesh=plsc.VectorSubcoreMesh(**_MESH),
        scratch_types=[
            pltpu.VMEM((80,), jnp.int32),
            pltpu.VMEM((C + 16,), jnp.int32),
            pltpu.VMEM((C * 16 + 16,), jnp.float32),
            pltpu.VMEM((NT * 16,), jnp.float32),
        ],
    )
    def prep(dsts, ae16, eb, out, eb_v, dst_v, ae_v, acc):
        wid = lax.axis_index("s") * 2 + lax.axis_index("c")
        pltpu.sync_copy(eb, eb_v)
        zero16 = jnp.zeros((16,), jnp.float32)
        for sub in range(2):
            b = wid * 2 + sub
            nbase = b * NT

            def zbody(i, _):
                acc[pl.ds(i * 16, 16)] = zero16
                return 0

            lax.fori_loop(0, NT, zbody, 0)
            ebv = eb_v[pl.ds(b, 16)]
            e0 = ebv[0]
            e1 = ebv[1]

            def cbody(j, _):
                cb = j * C
                pltpu.sync_copy(dsts.at[pl.ds(cb, C)], dst_v.at[pl.ds(0, C)])
                pltpu.sync_copy(ae16.at[pl.ds(cb * 16, C * 16)],
                                ae_v.at[pl.ds(0, C * 16)])
                ks = jnp.maximum(e0 - cb, 0)
                ke = jnp.minimum(e1 - cb, C)

                def ebody(k, _):
                    dk = dst_v[pl.ds(k, 16)][0]
                    dl = jnp.clip(dk - nbase, 0, NT - 1)
                    cur = acc[pl.ds(dl * 16, 16)]
                    acc[pl.ds(dl * 16, 16)] = cur + ae_v[pl.ds(k * 16, 16)]
                    return 0

                lax.fori_loop(ks, ke, ebody, 0)
                return 0

            lax.fori_loop(e0 // C, (e1 + C - 1) // C, cbody, 0)
            pltpu.sync_copy(acc, out.at[pl.ds(nbase * 16, NT * 16)])

    return prep


@functools.cache
def _make_main(l):
    @functools.partial(
        pl.kernel,
        out_type=jax.ShapeDtypeStruct((N_PAD * 80,), jnp.float32),
        mesh=plsc.VectorSubcoreMesh(**_MESH),
        compiler_params=pltpu.CompilerParams(use_tc_tiling_on_sc=False),
        scratch_types=[
            pltpu.VMEM((80,), jnp.int32),
            pltpu.VMEM((2 * C,), jnp.int32),
            pltpu.VMEM((2 * C + 16,), jnp.int32),
            pltpu.VMEM((2 * C * 16 + 16,), jnp.float32),
            pltpu.VMEM((2 * C, 80), jnp.float32),
            pltpu.VMEM((NT * 4 + 16,), jnp.float32),
            pltpu.VMEM((NT * 80,), jnp.float32),
            pltpu.SemaphoreType.DMA,
            pltpu.SemaphoreType.DMA,
            pltpu.SemaphoreType.DMA,
            pltpu.SemaphoreType.DMA,
        ],
    )
    def main(node_tab, ad_tab, srcs, dsts, ae16, eb, out,
             eb_v, idx2, dst2, ae2, rows2, ad_v, acc,
             sga, sgb, sla, slb):
        wid = lax.axis_index("s") * 2 + lax.axis_index("c")
        pltpu.sync_copy(eb, eb_v)
        zero16 = jnp.zeros((16,), jnp.float32)
        sg = (sga, sgb)
        sl = (sla, slb)

        def start(j, o):
            cb = j * C
            pltpu.sync_copy(srcs.at[pl.ds(cb, C)], idx2.at[pl.ds(o * C, C)])
            pltpu.async_copy(node_tab.at[idx2.at[pl.ds(o * C, C)]],
                             rows2.at[pl.ds(o * C, C)], sg[o])
            pltpu.async_copy(dsts.at[pl.ds(cb, C)],
                             dst2.at[pl.ds(o * C, C)], sl[o])
            pltpu.async_copy(ae16.at[pl.ds(cb * 16, C * 16)],
                             ae2.at[pl.ds(o * C * 16, C * 16)], sl[o])

        def wait_slot(o):
            pltpu.make_async_copy(node_tab.at[idx2.at[pl.ds(o * C, C)]],
                                  rows2.at[pl.ds(o * C, C)], sg[o]).wait()
            pltpu.make_async_copy(dsts.at[pl.ds(0, C)],
                                  dst2.at[pl.ds(o * C, C)], sl[o]).wait()
            pltpu.make_async_copy(ae16.at[pl.ds(0, C * 16)],
                                  ae2.at[pl.ds(o * C * 16, C * 16)], sl[o]).wait()

        for sub in range(2):
            b = wid * 2 + sub
            nbase = b * NT
            pltpu.sync_copy(ad_tab.at[pl.ds(nbase * 4, NT * 4)],
                            ad_v.at[pl.ds(0, NT * 4)])

            def zacc(i, _):
                acc[pl.ds(i * 16, 16)] = zero16
                return 0

            lax.fori_loop(0, NT * 5, zacc, 0)
            ebv = eb_v[pl.ds(b, 16)]
            e0 = ebv[0]
            e1 = ebv[1]
            c0 = e0 // C
            c1 = (e1 + C - 1) // C

            def process(j, o):
                cb = j * C
                ks = jnp.maximum(e0 - cb, 0)
                ke = jnp.minimum(e1 - cb, C)

                def ebody(k, _):
                    dk = dst2[pl.ds(o * C + k, 16)][0]
                    dl = jnp.clip(dk - nbase, 0, NT - 1)
                    asr = rows2[o * C + k, pl.ds(64, 16)]
                    adv = ad_v[pl.ds(dl * 4, 16)]
                    aev = ae2[pl.ds((o * C + k) * 16 + 4 * l, 16)]
                    a = asr + adv + aev
                    ex = jnp.exp(jnp.maximum(a, 0.2 * a))
                    dbase = dl * 80
                    cur = acc[pl.ds(dbase + 64, 16)]
                    acc[pl.ds(dbase + 64, 16)] = cur + ex
                    for h in range(HEADS):
                        exs = ex[h]
                        row = rows2[o * C + k, pl.ds(h * 16, 16)]
                        c2 = acc[pl.ds(dbase + h * 16, 16)]
                        acc[pl.ds(dbase + h * 16, 16)] = c2 + exs * row
                    return 0

                lax.fori_loop(ks, ke, ebody, 0)

            @pl.when(c1 > c0)
            def _():
                start(c0, 0)

            def pair_body(i, _):
                j0 = c0 + 2 * i
                j1 = j0 + 1

                @pl.when(j1 < c1)
                def _():
                    start(j1, 1)

                wait_slot(0)
                process(j0, 0)

                @pl.when(j1 + 1 < c1)
                def _():
                    start(j1 + 1, 0)

                @pl.when(j1 < c1)
                def _():
                    wait_slot(1)
                    process(j1, 1)

                return 0

            lax.fori_loop(0, (c1 - c0 + 1) // 2, pair_body, 0)
            pltpu.sync_copy(acc, out.at[pl.ds(nbase * 80, NT * 80)])

    return main


@functools.cache
def _make_perm():
    @functools.partial(
        pl.kernel,
        out_type=jax.ShapeDtypeStruct((E, 8), jnp.int32),
        mesh=plsc.VectorSubcoreMesh(**_MESH),
        compiler_params=pltpu.CompilerParams(use_tc_tiling_on_sc=False),
        scratch_types=[
            pltpu.VMEM((2 * C,), jnp.int32),
            pltpu.VMEM((2 * C, 8), jnp.int32),
            pltpu.SemaphoreType.DMA,
            pltpu.SemaphoreType.DMA,
        ],
    )
    def permk(pk, perm, out, idx2, rows2, sga, sgb):
        wid = lax.axis_index("s") * 2 + lax.axis_index("c")
        per = E // 32
        e0 = wid * per
        c0 = e0 // C
        c1 = (e0 + per + C - 1) // C
        sg = (sga, sgb)

        def start(j, o):
            cb = j * C
            pltpu.sync_copy(perm.at[pl.ds(cb, C)], idx2.at[pl.ds(o * C, C)])
            pltpu.async_copy(pk.at[idx2.at[pl.ds(o * C, C)]],
                             rows2.at[pl.ds(o * C, C)], sg[o])

        def fin(j, o):
            pltpu.make_async_copy(pk.at[idx2.at[pl.ds(o * C, C)]],
                                  rows2.at[pl.ds(o * C, C)], sg[o]).wait()
            pltpu.sync_copy(rows2.at[pl.ds(o * C, C)], out.at[pl.ds(j * C, C)])

        start(c0, 0)

        def pair_body(i, _):
            j0 = c0 + 2 * i
            j1 = j0 + 1

            @pl.when(j1 < c1)
            def _():
                start(j1, 1)

            fin(j0, 0)

            @pl.when(j1 + 1 < c1)
            def _():
                start(j1 + 1, 0)

            @pl.when(j1 < c1)
            def _():
                fin(j1, 1)

            return 0

        lax.fori_loop(0, (c1 - c0 + 1) // 2, pair_body, 0)

    return permk


@functools.cache
def _make_pool():
    @functools.partial(
        pl.kernel,
        out_type=[jax.ShapeDtypeStruct((G * 64,), jnp.float32),
                  jax.ShapeDtypeStruct((G * 64,), jnp.float32)],
        mesh=plsc.VectorSubcoreMesh(**_MESH),
        scratch_types=[
            pltpu.VMEM((144,), jnp.int32),
            pltpu.VMEM((C * 64,), jnp.float32),
            pltpu.VMEM((64,), jnp.float32),
            pltpu.VMEM((64,), jnp.float32),
        ],
    )
    def pool(h_pad, gb, gsum, gmax, gb_v, h_v, ssum, smax):
        wid = lax.axis_index("s") * 2 + lax.axis_index("c")
        pltpu.sync_copy(gb, gb_v)
        zero16 = jnp.zeros((16,), jnp.float32)
        ninf16 = jnp.full((16,), -1e30, jnp.float32)
        for gg in range(4):
            g = wid * 4 + gg
            gbv = gb_v[pl.ds(g, 16)]
            s = gbv[0]
            e = gbv[1]
            for h in range(4):
                ssum[pl.ds(h * 16, 16)] = zero16
                smax[pl.ds(h * 16, 16)] = ninf16

            def cbody(j, _):
                cb = j * C
                pltpu.sync_copy(h_pad.at[pl.ds(cb * 64, C * 64)], h_v)
                ks = jnp.maximum(s - cb, 0)
                ke = jnp.minimum(e - cb, C)

                def nbody(k, _):
                    for h in range(4):
                        r = h_v[pl.ds(k * 64 + h * 16, 16)]
                        cs = ssum[pl.ds(h * 16, 16)]
                        ssum[pl.ds(h * 16, 16)] = cs + r
                        cm = smax[pl.ds(h * 16, 16)]
                        smax[pl.ds(h * 16, 16)] = jnp.maximum(cm, r)
                    return 0

                lax.fori_loop(ks, ke, nbody, 0)
                return 0

            lax.fori_loop(s // C, (e + C - 1) // C, cbody, 0)
            pltpu.sync_copy(ssum, gsum.at[pl.ds(g * 64, 64)])
            pltpu.sync_copy(smax, gmax.at[pl.ds(g * 64, 64)])

    return pool


def kernel(x, edge_index, edge_attr, batch, global_features, params):
    src = edge_index[0]
    dst = edge_index[1]
    perm = jnp.argsort(dst)
    pk = jnp.concatenate([
        src[:, None], dst[:, None],
        jax.lax.bitcast_convert_type(edge_attr, jnp.int32),
        jnp.zeros((E, 3), jnp.int32)], axis=1)
    pks = _make_perm()(pk, perm)
    src_s = pks[:, 0]
    dst_s = pks[:, 1]

    nb = jnp.searchsorted(
        dst_s, jnp.arange(N_PAD + 1, dtype=jnp.int32)).astype(jnp.int32)
    deg = (nb[1:N + 1] - nb[:N]).astype(jnp.float32)
    eb = jnp.pad(nb[jnp.arange(NBLK + 1) * NT], (0, 80 - (NBLK + 1)),
                 constant_values=E)

    Vs_list, Vd_list, M_list, c_list = [], [], [], []
    for lp in params['layers']:
        Wr = lp['W'].reshape(HID, HEADS, CH)
        Vs_list.append(jnp.einsum('dhc,hc->dh', Wr, lp['att_src']))
        Vd_list.append(jnp.einsum('dhc,hc->dh', Wr, lp['att_dst']))
        Wer = lp['W_edge'].reshape(HID, HEADS, CH)
        Ve = jnp.einsum('dhc,hc->dh', Wer, lp['att_edge'])
        M_list.append(params['edge_W'] @ Ve)
        c_list.append(params['edge_b'] @ Ve)
    Mcat = jnp.concatenate(M_list, axis=1)
    M8 = jnp.zeros((8, 16), jnp.float32).at[2:5].set(Mcat)
    ccat = jnp.concatenate(c_list).reshape(1, 16)

    ae16 = pl.pallas_call(
        _aedge_body,
        grid=(250,),
        in_specs=[pl.BlockSpec((3200, 8), lambda i: (i, 0)),
                  pl.BlockSpec((8, 16), lambda i: (0, 0)),
                  pl.BlockSpec((1, 16), lambda i: (0, 0))],
        out_specs=pl.BlockSpec((3200, 16), lambda i: (i, 0)),
        out_shape=jax.ShapeDtypeStruct((E, 16), jnp.float32),
    )(pks, M8, ccat)

    esum = _make_prep()(dst_s, ae16.reshape(-1), eb).reshape(N_PAD, 16)
    el16 = esum[:N] / jnp.maximum(deg, 1.0)[:, None]

    x8 = jnp.pad(x, ((0, 0), (0, 1)))
    nw8 = jnp.pad(params['node_W'], ((0, 1), (0, 0)))
    h = pl.pallas_call(
        _embed_body,
        grid=(25,),
        in_specs=[pl.BlockSpec((2000, 8), lambda i: (i, 0)),
                  pl.BlockSpec((8, 64), lambda i: (0, 0)),
                  pl.BlockSpec((1, 64), lambda i: (0, 0))],
        out_specs=pl.BlockSpec((2000, 64), lambda i: (i, 0)),
        out_shape=jax.ShapeDtypeStruct((N, 64), jnp.float32),
    )(x8, nw8, params['node_b'].reshape(1, 64))

    for l, lp in enumerate(params['layers']):
        Vsd = jnp.concatenate([Vs_list[l], Vd_list[l]], axis=1)
        node_tab, a_dst = pl.pallas_call(
            _pre_body,
            grid=(25,),
            in_specs=[pl.BlockSpec((2000, 64), lambda i: (i, 0)),
                      pl.BlockSpec((64, 64), lambda i: (0, 0)),
                      pl.BlockSpec((64, 8), lambda i: (0, 0))],
            out_specs=[pl.BlockSpec((2000, 80), lambda i: (i, 0)),
                       pl.BlockSpec((2000, 4), lambda i: (i, 0))],
            out_shape=[jax.ShapeDtypeStruct((N, 80), jnp.float32),
                       jax.ShapeDtypeStruct((N, 4), jnp.float32)],
        )(h, lp['W'], Vsd)
        ad_pad = jnp.pad(a_dst, ((0, N_PAD - N), (0, 0)))
        out_tab = _make_main(l)(node_tab, ad_pad.reshape(-1), src_s, dst_s,
                                ae16.reshape(-1), eb).reshape(N_PAD, 80)
        el_l = el16[:, 4 * l:4 * l + 4]
        h = pl.pallas_call(
            _epi_body,
            grid=(2, 25),
            in_specs=[pl.BlockSpec((2000, 80), lambda p, i: (i, 0)),
                      pl.BlockSpec((2000, 80), lambda p, i: (i, 0)),
                      pl.BlockSpec((2000, 4), lambda p, i: (i, 0)),
                      pl.BlockSpec((2000, 4), lambda p, i: (i, 0)),
                      pl.BlockSpec((2000, 64), lambda p, i: (i, 0)),
                      pl.BlockSpec((1, 64), lambda p, i: (0, 0)),
                      pl.BlockSpec((1, 64), lambda p, i: (0, 0)),
                      pl.BlockSpec((1, 64), lambda p, i: (0, 0))],
            out_specs=pl.BlockSpec((2000, 64), lambda p, i: (i, 0)),
            out_shape=jax.ShapeDtypeStruct((N, 64), jnp.float32),
            scratch_shapes=[pltpu.VMEM((1, 64), jnp.float32),
                            pltpu.VMEM((1, 64), jnp.float32)],
        )(out_tab[:N], node_tab, a_dst, el_l, h,
          lp['bias'].reshape(1, 64), lp['bn_gamma'].reshape(1, 64),
          lp['bn_beta'].reshape(1, 64))

    gb = jnp.searchsorted(
        batch, jnp.arange(G + 1, dtype=jnp.int32)).astype(jnp.int32)
    gb_pad = jnp.pad(gb, (0, 144 - (G + 1)), constant_values=N)
    counts = (gb[1:] - gb[:-1]).astype(jnp.float32).reshape(G, 1)
    h_pad = jnp.pad(h, ((0, N_PAD - N), (0, 0)))
    gsum3, gmax3 = _make_pool()(h_pad.reshape(-1), gb_pad)

    out = pl.pallas_call(
        _head_body,
        out_shape=jax.ShapeDtypeStruct((G, 5), jnp.float32),
    )(gsum3.reshape(G, 64), gmax3.reshape(G, 64), counts, global_features,
      params['gc_W'], params['gc_b'].reshape(1, 64),
      params['gf1_W'], params['gf1_b'].reshape(1, 32),
      params['gf2_W'], params['gf2_b'].reshape(1, 32),
      params['p1_W'], params['p1_b'].reshape(1, 64),
      params['p2_W'], params['p2_b'].reshape(1, 32),
      params['p3_W'], params['p3_b'].reshape(1, 5))
    return out
